# baseline (speedup 1.0000x reference)
import functools

import jax
import jax.numpy as jnp
from jax import lax
from jax.experimental import pallas as pl
from jax.experimental.pallas import tpu as pltpu
from jax.experimental.pallas import tpu_sc as plsc

_N_HASHES = 4
_CHANNELS = 64
_REDUCTION = 4
_CHUNK = 144
_HW = 224
_L = _HW * _HW
_PAD = (_CHUNK - _L % _CHUNK) % _CHUNK
_LP = _L + _PAD
_NCH = _LP // _CHUNK
_CM = _CHANNELS // _REDUCTION
_NB = 128


_NE = 352
_CB = 88
_NBLK = _NE // _CB


def _norm_keys(xx):
    nrm = jnp.sqrt(jnp.sum(xx * xx, axis=1, keepdims=True))
    return xx / jnp.maximum(nrm, 5e-5)


def _attn_kernel(x_ref, xp_ref, xn_ref, x0_ref, x347_ref, x348_ref,
                 y_ref, yp_ref, yn_ref, y0_ref, y347_ref, y348_ref,
                 out_ref, bs_ref, kn_ref):
    cb = pl.program_id(1)
    is_last = cb == _NBLK - 1
    is_first = cb == 0
    jwrap = _NCH - 1 - (_NBLK - 1) * _CB

    c348x = jnp.concatenate(
        [x348_ref[0, 0, :_L % _CHUNK], x347_ref[0, 0, _CHUNK - _PAD + 64:],
         x348_ref[0, 0, :_L % _CHUNK]], axis=0)
    c348y = jnp.concatenate(
        [y348_ref[0, 0, :_L % _CHUNK], y347_ref[0, 0, _CHUNK - _PAD + 64:],
         y348_ref[0, 0, :_L % _CHUNK]], axis=0)

    kn_ref[0] = _norm_keys(
        jnp.where(is_first, c348x, xp_ref[0, 0])).astype(jnp.bfloat16)
    kn_ref[_CB + 1] = _norm_keys(xn_ref[0, 0]).astype(jnp.bfloat16)

    def nbody(j, carry):
        kn_ref[j + 1] = _norm_keys(x_ref[0, j]).astype(jnp.bfloat16)
        return carry

    lax.fori_loop(0, _CB, nbody, 0)

    @pl.when(is_last)
    def _():
        kn_ref[jwrap + 1] = _norm_keys(c348x).astype(jnp.bfloat16)
        kn_ref[jwrap + 2] = _norm_keys(x0_ref[0, 0]).astype(jnp.bfloat16)

    def body(j, carry):
        jm = jnp.maximum(j - 1, 0)
        jp = jnp.minimum(j + 1, _CB - 1)
        sel_wrap = is_last & (j == jwrap)
        q = jnp.where(sel_wrap, c348x, x_ref[0, j])
        ys = jnp.where(sel_wrap, c348y, y_ref[0, j])
        yprev = jnp.where(j == 0,
                          jnp.where(is_first, c348y, yp_ref[0, 0]),
                          y_ref[0, jm])
        ynext = jnp.where(is_last & (j == jwrap - 1), c348y,
                          jnp.where(sel_wrap, y0_ref[0, 0],
                                    jnp.where(j == _CB - 1, yn_ref[0, 0],
                                              y_ref[0, jp])))

        kcat = jnp.concatenate(
            [kn_ref[j + 1], kn_ref[j], kn_ref[j + 2]],
            axis=0)
        raw = lax.dot_general(
            q.astype(jnp.bfloat16), kcat,
            (((1,), (1,)), ((), ())),
            preferred_element_type=jnp.float32)
        m = jnp.max(raw, axis=1)
        e = jnp.exp(raw - m[:, None])
        s = jnp.sum(e, axis=1)
        p = (e * (1.0 / s)[:, None]).astype(jnp.bfloat16)
        ycat = jnp.concatenate([ys, yprev, ynext], axis=0)
        out_ref[0, j] = lax.dot_general(
            p, ycat, (((1,), (0,)), ((), ())),
            preferred_element_type=jnp.float32)
        bs_ref[0, j] = jnp.log(s) + m
        return carry

    lax.fori_loop(0, _CB, body, 0, unroll=2)


def _chunked_attention(x4, y4):
    G = x4.shape[0]

    def _prev(g, cb):
        return (g, jnp.maximum(cb * _CB - 1, 0), 0, 0)

    def _next(g, cb):
        return (g, jnp.minimum(cb * _CB + _CB, _NCH - 1), 0, 0)

    halo_x = [
        pl.BlockSpec((1, 1, _CHUNK, _CM), _prev),
        pl.BlockSpec((1, 1, _CHUNK, _CM), _next),
        pl.BlockSpec((1, 1, _CHUNK, _CM), lambda g, cb: (g, 0, 0, 0)),
        pl.BlockSpec((1, 1, _CHUNK, _CM), lambda g, cb: (g, _NCH - 2, 0, 0)),
        pl.BlockSpec((1, 1, _CHUNK, _CM), lambda g, cb: (g, _NCH - 1, 0, 0)),
    ]
    halo_y = [
        pl.BlockSpec((1, 1, _CHUNK, _CHANNELS), _prev),
        pl.BlockSpec((1, 1, _CHUNK, _CHANNELS), _next),
        pl.BlockSpec((1, 1, _CHUNK, _CHANNELS), lambda g, cb: (g, 0, 0, 0)),
        pl.BlockSpec((1, 1, _CHUNK, _CHANNELS),
                     lambda g, cb: (g, _NCH - 2, 0, 0)),
        pl.BlockSpec((1, 1, _CHUNK, _CHANNELS),
                     lambda g, cb: (g, _NCH - 1, 0, 0)),
    ]
    return pl.pallas_call(
        _attn_kernel,
        grid=(G, _NBLK),
        in_specs=(
            [pl.BlockSpec((1, _CB, _CHUNK, _CM),
                          lambda g, cb: (g, cb, 0, 0))] + halo_x
            + [pl.BlockSpec((1, _CB, _CHUNK, _CHANNELS),
                            lambda g, cb: (g, cb, 0, 0))] + halo_y),
        out_specs=[
            pl.BlockSpec((1, _CB, _CHUNK, _CHANNELS),
                         lambda g, cb: (g, cb, 0, 0)),
            pl.BlockSpec((1, _CB, _CHUNK), lambda g, cb: (g, cb, 0)),
        ],
        out_shape=[
            jax.ShapeDtypeStruct((G, _NCH, _CHUNK, _CHANNELS), jnp.float32),
            jax.ShapeDtypeStruct((G, _NCH, _CHUNK), jnp.float32),
        ],
        scratch_shapes=[pltpu.VMEM((_CB + 2, _CHUNK, _CM), jnp.bfloat16)],
    )(x4, x4, x4, x4, x4, x4, y4, y4, y4, y4, y4, y4)


_K = 512
_NSLC = 49
_BLKE = _K * _NSLC
_NBLK_S = (_N_HASHES * _L) // _BLKE


def _hist_kernel(codes_ref, h_ref):
    acc = jnp.zeros((_K, 64), jnp.int32)
    bcol = lax.broadcasted_iota(jnp.int32, (_K, _K), 0)
    lane = lax.broadcasted_iota(jnp.int32, (_K, 64), 1)
    for s in range(_NSLC):
        c = codes_ref[0, 0, pl.ds(s * _K, _K)]
        e2 = (bcol == jnp.broadcast_to(c[None, :], (_K, _K))).astype(jnp.int32)
        hs = jnp.sum(e2, axis=1, keepdims=True)
        acc = acc + jnp.where(lane == s, hs, 0)
    h_ref[0, 0] = acc


def _rank_kernel(codes_ref, start_ref, dest_ref):
    bcol = lax.broadcasted_iota(jnp.int32, (_K, _K), 0)
    utri = (bcol <= lax.broadcasted_iota(jnp.int32, (_K, _K), 1))
    utri = utri.astype(jnp.bfloat16)
    for s in range(_NSLC):
        c = codes_ref[0, 0, pl.ds(s * _K, _K)]
        e2f = (bcol == jnp.broadcast_to(c[None, :], (_K, _K)))
        e2f = e2f.astype(jnp.float32)
        cum = lax.dot_general(e2f.astype(jnp.bfloat16), utri,
                              (((1,), (0,)), ((), ())),
                              preferred_element_type=jnp.float32)
        rank = jnp.sum(e2f * (cum - e2f), axis=0)
        scol = start_ref[0, 0, :, s:s + 1]
        ssel = jnp.sum(e2f * scol, axis=0)
        dest_ref[0, 0, pl.ds(s * _K, _K)] = (rank + ssel).astype(jnp.int32)


def _counting_sort_dest(hash_codes):
    N = hash_codes.shape[0]
    NB = N * _NBLK_S
    codes = hash_codes.reshape(NB, 1, _BLKE)
    h = pl.pallas_call(
        _hist_kernel,
        grid=(NB,),
        in_specs=[pl.BlockSpec((1, 1, _BLKE), lambda b: (b, 0, 0))],
        out_specs=pl.BlockSpec((1, 1, _K, 64), lambda b: (b, 0, 0, 0)),
        out_shape=jax.ShapeDtypeStruct((NB, 1, _K, 64), jnp.int32),
    )(codes)
    h_lin = h.reshape(N, _NBLK_S, _K, 64).transpose(0, 1, 3, 2)
    h_lin = h_lin[:, :, :_NSLC, :].reshape(N, _NBLK_S * _NSLC, _K)
    totals = jnp.sum(h_lin, axis=1)
    gstart = jnp.cumsum(totals, axis=1) - totals
    pref = jnp.cumsum(h_lin, axis=1) - h_lin
    start = gstart[:, None, :] + pref
    start = start.reshape(N, _NBLK_S, _NSLC, _K)
    start = jnp.pad(start, ((0, 0), (0, 0), (0, 64 - _NSLC), (0, 0)))
    start = start.transpose(0, 1, 3, 2).astype(jnp.float32)
    start = start.reshape(NB, 1, _K, 64)
    dest = pl.pallas_call(
        _rank_kernel,
        grid=(NB,),
        in_specs=[pl.BlockSpec((1, 1, _BLKE), lambda b: (b, 0, 0)),
                  pl.BlockSpec((1, 1, _K, 64), lambda b: (b, 0, 0, 0))],
        out_specs=pl.BlockSpec((1, 1, _BLKE), lambda b: (b, 0, 0)),
        out_shape=jax.ShapeDtypeStruct((NB, 1, _BLKE), jnp.int32),
    )(codes, start)
    return dest.reshape(N, _N_HASHES * _L)


_LSHB = 6272


def _lsh_kernel(x_ref, rot_ref, code_ref):
    scores = lax.dot_general(
        x_ref[0].astype(jnp.bfloat16), rot_ref[...].astype(jnp.bfloat16),
        (((1,), (0,)), ((), ())),
        preferred_element_type=jnp.float32)
    iota = lax.broadcasted_iota(jnp.int32, (_LSHB, _NB // 2), 1)
    for h in range(_N_HASHES):
        r = scores[:, h * (_NB // 2):(h + 1) * (_NB // 2)]
        m1 = jnp.max(r, axis=1, keepdims=True)
        m2 = -jnp.min(r, axis=1, keepdims=True)
        i1 = jnp.min(jnp.where(r == m1, iota, _NB), axis=1)
        i2 = jnp.min(jnp.where(-r == m2, iota + _NB // 2, _NB), axis=1)
        code = jnp.where(m1[:, 0] >= m2[:, 0], i1, i2) + h * _NB
        code_ref[0, h] = code


def _lsh_codes(x_embed, rotations):
    N = x_embed.shape[0]
    rot2 = rotations.reshape(_CM, _N_HASHES * (_NB // 2))
    nb = _L // _LSHB
    codes = pl.pallas_call(
        _lsh_kernel,
        grid=(N, nb),
        in_specs=[
            pl.BlockSpec((1, _LSHB, _CM), lambda n, b: (n, b, 0)),
            pl.BlockSpec((_CM, _N_HASHES * (_NB // 2)), lambda n, b: (0, 0)),
        ],
        out_specs=pl.BlockSpec((1, _N_HASHES, _LSHB),
                               lambda n, b: (n, 0, b)),
        out_shape=jax.ShapeDtypeStruct((N, _N_HASHES, _L), jnp.int32),
    )(x_embed, rot2)
    return codes.reshape(N, _N_HASHES * _L)


_SCCH = 448
_SCW = 32


def _sc_fwd_scatter(x_embed, y_embed, dest_flat):
    N = x_embed.shape[0]
    n4l = N * _N_HASHES * _L
    n_out = N * _N_HASHES * _LP
    nch = n4l // _SCCH
    per_tile = nch // _SCW
    cpb = (_N_HASHES * _L) // _SCCH
    cps = _L // _SCCH
    mesh = plsc.VectorSubcoreMesh(core_axis_name="c", subcore_axis_name="s")

    @functools.partial(
        pl.kernel, mesh=mesh,
        compiler_params=pltpu.CompilerParams(use_tc_tiling_on_sc=False),
        out_type=[jax.ShapeDtypeStruct((n_out, _CM), jnp.float32),
                  jax.ShapeDtypeStruct((n_out, _CHANNELS), jnp.bfloat16)],
        scratch_types=[pltpu.VMEM((_SCCH,), jnp.int32),
                       pltpu.VMEM((_SCCH, _CM), jnp.float32),
                       pltpu.VMEM((_SCCH, _CHANNELS), jnp.bfloat16),
                       pltpu.SemaphoreType.DMA,
                       pltpu.SemaphoreType.DMA])
    def scat(x_hbm, y_hbm, d_hbm, xs_hbm, ys_hbm, idx_v, xr_v, yr_v,
             sem_in, sem_out):
        wid = lax.axis_index("s") * 2 + lax.axis_index("c")

        @pl.loop(0, per_tile)
        def _(t):
            c = wid * per_tile + t
            n = c // cpb
            l0 = (c % cps) * _SCCH
            cp1 = pltpu.async_copy(d_hbm.at[pl.ds(c * _SCCH, _SCCH)],
                                   idx_v, sem_in)
            cp2 = pltpu.async_copy(x_hbm.at[n, pl.ds(l0, _SCCH)],
                                   xr_v, sem_in)
            cp3 = pltpu.async_copy(y_hbm.at[n, pl.ds(l0, _SCCH)],
                                   yr_v, sem_in)
            cp1.wait()
            cp2.wait()
            cp3.wait()
            cp4 = pltpu.async_copy(xr_v, xs_hbm.at[idx_v], sem_out)
            cp5 = pltpu.async_copy(yr_v, ys_hbm.at[idx_v], sem_out)
            cp4.wait()
            cp5.wait()

    return scat(x_embed, y_embed, dest_flat)


def _conv2d(x, w, b, pad):
    y = lax.conv_general_dilated(
        x, w, (1, 1), [(pad, pad), (pad, pad)],
        dimension_numbers=('NCHW', 'OIHW', 'NCHW'))
    return y + b[None, :, None, None]


def _bgather(v, idx):
    return v[jnp.arange(v.shape[0])[:, None], idx]


@jax.jit
def kernel(input, W_match, b_match, W_asm, b_asm, rotations):
    N, C, H, W = input.shape
    x_embed = _conv2d(input, W_match, b_match, 1).reshape(N, _CM, _L)
    x_embed = x_embed.transpose(0, 2, 1)
    y_embed = _conv2d(input, W_asm, b_asm, 0).reshape(N, C, _L)
    y_embed = y_embed.transpose(0, 2, 1)

    hash_codes = _lsh_codes(x_embed, rotations)

    undo_sort = _counting_sort_dest(hash_codes)
    undo_p = undo_sort + _PAD * (undo_sort // _L)
    dest_flat = (undo_p
                 + (jnp.arange(N, dtype=jnp.int32)
                    * (_N_HASHES * _LP))[:, None]).reshape(-1)
    x_sf, y_sf = _sc_fwd_scatter(x_embed, y_embed.astype(jnp.bfloat16),
                                 dest_flat)
    G = N * _N_HASHES
    x4 = x_sf.reshape(G, _NCH, _CHUNK, _CM)
    y4 = y_sf.reshape(G, _NCH, _CHUNK, C)

    ret4, bs4 = _chunked_attention(x4, y4)

    ret = ret4.reshape(N, _N_HASHES * _LP, C)
    bs = bs4.reshape(N, _N_HASHES * _LP)
    ret = _bgather(ret, undo_p)
    bs = jnp.take_along_axis(bs, undo_p, axis=1)
    ret = ret.reshape(N, _N_HASHES, _L, C)
    bs = bs.reshape(N, _N_HASHES, _L, 1)
    probs = jax.nn.softmax(bs, axis=1)
    ret = jnp.sum(ret * probs, axis=1)
    ret = ret.transpose(0, 2, 1).reshape(N, C, H, W) + input
    return ret


# speedup vs baseline: 1.1323x; 1.1323x over previous
import functools

import jax
import jax.numpy as jnp
from jax import lax
from jax.experimental import pallas as pl
from jax.experimental.pallas import tpu as pltpu
from jax.experimental.pallas import tpu_sc as plsc

_N_HASHES = 4
_CHANNELS = 64
_REDUCTION = 4
_CHUNK = 144
_HW = 224
_L = _HW * _HW
_PAD = (_CHUNK - _L % _CHUNK) % _CHUNK
_LP = _L + _PAD
_NCH = _LP // _CHUNK
_CM = _CHANNELS // _REDUCTION
_NB = 128


_NE = 352
_CB = 88
_NBLK = _NE // _CB


def _norm_keys(xx):
    nrm = jnp.sqrt(jnp.sum(xx * xx, axis=1, keepdims=True))
    return xx / jnp.maximum(nrm, 5e-5)


def _attn_kernel(x_ref, xp_ref, xn_ref, x0_ref, x347_ref, x348_ref,
                 y_ref, yp_ref, yn_ref, y0_ref, y347_ref, y348_ref,
                 out_ref, bs_ref, kn_ref):
    cb = pl.program_id(1)
    is_last = cb == _NBLK - 1
    is_first = cb == 0
    jwrap = _NCH - 1 - (_NBLK - 1) * _CB

    c348x = jnp.concatenate(
        [x348_ref[0, 0, :_L % _CHUNK], x347_ref[0, 0, _CHUNK - _PAD + 64:],
         x348_ref[0, 0, :_L % _CHUNK]], axis=0)
    c348y = jnp.concatenate(
        [y348_ref[0, 0, :_L % _CHUNK], y347_ref[0, 0, _CHUNK - _PAD + 64:],
         y348_ref[0, 0, :_L % _CHUNK]], axis=0)

    kn_ref[0] = _norm_keys(
        jnp.where(is_first, c348x, xp_ref[0, 0])).astype(jnp.bfloat16)
    kn_ref[_CB + 1] = _norm_keys(xn_ref[0, 0]).astype(jnp.bfloat16)

    def nbody(j, carry):
        kn_ref[j + 1] = _norm_keys(x_ref[0, j]).astype(jnp.bfloat16)
        return carry

    lax.fori_loop(0, _CB, nbody, 0)

    @pl.when(is_last)
    def _():
        kn_ref[jwrap + 1] = _norm_keys(c348x).astype(jnp.bfloat16)
        kn_ref[jwrap + 2] = _norm_keys(x0_ref[0, 0]).astype(jnp.bfloat16)

    def body(j, carry):
        jm = jnp.maximum(j - 1, 0)
        jp = jnp.minimum(j + 1, _CB - 1)
        sel_wrap = is_last & (j == jwrap)
        q = jnp.where(sel_wrap, c348x, x_ref[0, j])
        ys = jnp.where(sel_wrap, c348y, y_ref[0, j])
        yprev = jnp.where(j == 0,
                          jnp.where(is_first, c348y, yp_ref[0, 0]),
                          y_ref[0, jm])
        ynext = jnp.where(is_last & (j == jwrap - 1), c348y,
                          jnp.where(sel_wrap, y0_ref[0, 0],
                                    jnp.where(j == _CB - 1, yn_ref[0, 0],
                                              y_ref[0, jp])))

        kcat = jnp.concatenate(
            [kn_ref[j + 1], kn_ref[j], kn_ref[j + 2]],
            axis=0)
        raw = lax.dot_general(
            q.astype(jnp.bfloat16), kcat,
            (((1,), (1,)), ((), ())),
            preferred_element_type=jnp.float32)
        m = jnp.max(raw, axis=1)
        e = jnp.exp(raw - m[:, None])
        s = jnp.sum(e, axis=1)
        p = (e * (1.0 / s)[:, None]).astype(jnp.bfloat16)
        ycat = jnp.concatenate([ys, yprev, ynext], axis=0)
        out_ref[0, j] = lax.dot_general(
            p, ycat, (((1,), (0,)), ((), ())),
            preferred_element_type=jnp.float32)
        bs_ref[0, j] = jnp.log(s) + m
        return carry

    lax.fori_loop(0, _CB, body, 0, unroll=4)


def _chunked_attention(x4, y4):
    G = x4.shape[0]

    def _prev(g, cb):
        return (g, jnp.maximum(cb * _CB - 1, 0), 0, 0)

    def _next(g, cb):
        return (g, jnp.minimum(cb * _CB + _CB, _NCH - 1), 0, 0)

    halo_x = [
        pl.BlockSpec((1, 1, _CHUNK, _CM), _prev),
        pl.BlockSpec((1, 1, _CHUNK, _CM), _next),
        pl.BlockSpec((1, 1, _CHUNK, _CM), lambda g, cb: (g, 0, 0, 0)),
        pl.BlockSpec((1, 1, _CHUNK, _CM), lambda g, cb: (g, _NCH - 2, 0, 0)),
        pl.BlockSpec((1, 1, _CHUNK, _CM), lambda g, cb: (g, _NCH - 1, 0, 0)),
    ]
    halo_y = [
        pl.BlockSpec((1, 1, _CHUNK, _CHANNELS), _prev),
        pl.BlockSpec((1, 1, _CHUNK, _CHANNELS), _next),
        pl.BlockSpec((1, 1, _CHUNK, _CHANNELS), lambda g, cb: (g, 0, 0, 0)),
        pl.BlockSpec((1, 1, _CHUNK, _CHANNELS),
                     lambda g, cb: (g, _NCH - 2, 0, 0)),
        pl.BlockSpec((1, 1, _CHUNK, _CHANNELS),
                     lambda g, cb: (g, _NCH - 1, 0, 0)),
    ]
    return pl.pallas_call(
        _attn_kernel,
        grid=(G, _NBLK),
        in_specs=(
            [pl.BlockSpec((1, _CB, _CHUNK, _CM),
                          lambda g, cb: (g, cb, 0, 0))] + halo_x
            + [pl.BlockSpec((1, _CB, _CHUNK, _CHANNELS),
                            lambda g, cb: (g, cb, 0, 0))] + halo_y),
        out_specs=[
            pl.BlockSpec((1, _CB, _CHUNK, _CHANNELS),
                         lambda g, cb: (g, cb, 0, 0)),
            pl.BlockSpec((1, _CB, _CHUNK), lambda g, cb: (g, cb, 0)),
        ],
        out_shape=[
            jax.ShapeDtypeStruct((G, _NCH, _CHUNK, _CHANNELS), jnp.float32),
            jax.ShapeDtypeStruct((G, _NCH, _CHUNK), jnp.float32),
        ],
        scratch_shapes=[pltpu.VMEM((_CB + 2, _CHUNK, _CM), jnp.bfloat16)],
    )(x4, x4, x4, x4, x4, x4, y4, y4, y4, y4, y4, y4)


_K = 512
_NSLC = 49
_BLKE = _K * _NSLC
_NBLK_S = (_N_HASHES * _L) // _BLKE


def _hist_kernel(codes_ref, h_ref):
    acc = jnp.zeros((_K, 64), jnp.int32)
    bcol = lax.broadcasted_iota(jnp.int32, (_K, _K), 0)
    lane = lax.broadcasted_iota(jnp.int32, (_K, 64), 1)
    for s in range(_NSLC):
        c = codes_ref[0, 0, pl.ds(s * _K, _K)]
        e2 = (bcol == jnp.broadcast_to(c[None, :], (_K, _K))).astype(jnp.int32)
        hs = jnp.sum(e2, axis=1, keepdims=True)
        acc = acc + jnp.where(lane == s, hs, 0)
    h_ref[0, 0] = acc


def _rank_kernel(codes_ref, start_ref, dest_ref):
    bcol = lax.broadcasted_iota(jnp.int32, (_K, _K), 0)
    utri = (bcol <= lax.broadcasted_iota(jnp.int32, (_K, _K), 1))
    utri = utri.astype(jnp.bfloat16)
    for s in range(_NSLC):
        c = codes_ref[0, 0, pl.ds(s * _K, _K)]
        e2f = (bcol == jnp.broadcast_to(c[None, :], (_K, _K)))
        e2f = e2f.astype(jnp.float32)
        cum = lax.dot_general(e2f.astype(jnp.bfloat16), utri,
                              (((1,), (0,)), ((), ())),
                              preferred_element_type=jnp.float32)
        rank = jnp.sum(e2f * (cum - e2f), axis=0)
        scol = start_ref[0, 0, :, s:s + 1]
        ssel = jnp.sum(e2f * scol, axis=0)
        dest_ref[0, 0, pl.ds(s * _K, _K)] = (rank + ssel).astype(jnp.int32)


def _counting_sort_dest(hash_codes):
    N = hash_codes.shape[0]
    NB = N * _NBLK_S
    codes = hash_codes.reshape(NB, 1, _BLKE)
    h = pl.pallas_call(
        _hist_kernel,
        grid=(NB,),
        in_specs=[pl.BlockSpec((1, 1, _BLKE), lambda b: (b, 0, 0))],
        out_specs=pl.BlockSpec((1, 1, _K, 64), lambda b: (b, 0, 0, 0)),
        out_shape=jax.ShapeDtypeStruct((NB, 1, _K, 64), jnp.int32),
    )(codes)
    h_lin = h.reshape(N, _NBLK_S, _K, 64).transpose(0, 1, 3, 2)
    h_lin = h_lin[:, :, :_NSLC, :].reshape(N, _NBLK_S * _NSLC, _K)
    totals = jnp.sum(h_lin, axis=1)
    gstart = jnp.cumsum(totals, axis=1) - totals
    pref = jnp.cumsum(h_lin, axis=1) - h_lin
    start = gstart[:, None, :] + pref
    start = start.reshape(N, _NBLK_S, _NSLC, _K)
    start = jnp.pad(start, ((0, 0), (0, 0), (0, 64 - _NSLC), (0, 0)))
    start = start.transpose(0, 1, 3, 2).astype(jnp.float32)
    start = start.reshape(NB, 1, _K, 64)
    dest = pl.pallas_call(
        _rank_kernel,
        grid=(NB,),
        in_specs=[pl.BlockSpec((1, 1, _BLKE), lambda b: (b, 0, 0)),
                  pl.BlockSpec((1, 1, _K, 64), lambda b: (b, 0, 0, 0))],
        out_specs=pl.BlockSpec((1, 1, _BLKE), lambda b: (b, 0, 0)),
        out_shape=jax.ShapeDtypeStruct((NB, 1, _BLKE), jnp.int32),
    )(codes, start)
    return dest.reshape(N, _N_HASHES * _L)


_SCCH = 448
_SCW = 32


def _sc_fwd_scatter(x_embed, y_embed, dest_flat):
    N = x_embed.shape[0]
    n4l = N * _N_HASHES * _L
    n_out = N * _N_HASHES * _LP
    nch = n4l // _SCCH
    per_tile = nch // _SCW
    cpb = (_N_HASHES * _L) // _SCCH
    cps = _L // _SCCH
    mesh = plsc.VectorSubcoreMesh(core_axis_name="c", subcore_axis_name="s")

    @functools.partial(
        pl.kernel, mesh=mesh,
        compiler_params=pltpu.CompilerParams(use_tc_tiling_on_sc=False),
        out_type=[jax.ShapeDtypeStruct((n_out, _CM), jnp.float32),
                  jax.ShapeDtypeStruct((n_out, _CHANNELS), jnp.bfloat16)],
        scratch_types=[pltpu.VMEM((_SCCH,), jnp.int32),
                       pltpu.VMEM((_SCCH, _CM), jnp.float32),
                       pltpu.VMEM((_SCCH, _CHANNELS), jnp.bfloat16),
                       pltpu.SemaphoreType.DMA,
                       pltpu.SemaphoreType.DMA])
    def scat(x_hbm, y_hbm, d_hbm, xs_hbm, ys_hbm, idx_v, xr_v, yr_v,
             sem_in, sem_out):
        wid = lax.axis_index("s") * 2 + lax.axis_index("c")

        @pl.loop(0, per_tile)
        def _(t):
            c = wid * per_tile + t
            n = c // cpb
            l0 = (c % cps) * _SCCH
            cp1 = pltpu.async_copy(d_hbm.at[pl.ds(c * _SCCH, _SCCH)],
                                   idx_v, sem_in)
            cp2 = pltpu.async_copy(x_hbm.at[n, pl.ds(l0, _SCCH)],
                                   xr_v, sem_in)
            cp3 = pltpu.async_copy(y_hbm.at[n, pl.ds(l0, _SCCH)],
                                   yr_v, sem_in)
            cp1.wait()
            cp2.wait()
            cp3.wait()
            cp4 = pltpu.async_copy(xr_v, xs_hbm.at[idx_v], sem_out)
            cp5 = pltpu.async_copy(yr_v, ys_hbm.at[idx_v], sem_out)
            cp4.wait()
            cp5.wait()

    return scat(x_embed, y_embed, dest_flat)


def _conv2d(x, w, b, pad):
    y = lax.conv_general_dilated(
        x, w, (1, 1), [(pad, pad), (pad, pad)],
        dimension_numbers=('NCHW', 'OIHW', 'NCHW'))
    return y + b[None, :, None, None]


def _bgather(v, idx):
    return v[jnp.arange(v.shape[0])[:, None], idx]


@jax.jit
def kernel(input, W_match, b_match, W_asm, b_asm, rotations):
    N, C, H, W = input.shape
    x_embed = _conv2d(input, W_match, b_match, 1).reshape(N, _CM, _L)
    x_embed = x_embed.transpose(0, 2, 1)
    y_embed = _conv2d(input, W_asm, b_asm, 0).reshape(N, C, _L)
    y_embed = y_embed.transpose(0, 2, 1)

    rotated = jnp.einsum('btf,fhi->bhti', x_embed, rotations)
    rotated = jnp.concatenate([rotated, -rotated], axis=-1)
    hash_codes = jnp.argmax(rotated, axis=-1)
    offsets = (jnp.arange(_N_HASHES) * _NB).reshape(1, -1, 1)
    hash_codes = (hash_codes + offsets).reshape(N, -1)

    undo_sort = _counting_sort_dest(hash_codes)
    undo_p = undo_sort + _PAD * (undo_sort // _L)
    dest_flat = (undo_p
                 + (jnp.arange(N, dtype=jnp.int32)
                    * (_N_HASHES * _LP))[:, None]).reshape(-1)
    x_sf, y_sf = _sc_fwd_scatter(x_embed, y_embed.astype(jnp.bfloat16),
                                 dest_flat)
    G = N * _N_HASHES
    x4 = x_sf.reshape(G, _NCH, _CHUNK, _CM)
    y4 = y_sf.reshape(G, _NCH, _CHUNK, C)

    ret4, bs4 = _chunked_attention(x4, y4)

    ret = ret4.reshape(N, _N_HASHES * _LP, C)
    bs = bs4.reshape(N, _N_HASHES * _LP)
    ret = _bgather(ret, undo_p)
    bs = jnp.take_along_axis(bs, undo_p, axis=1)
    ret = ret.reshape(N, _N_HASHES, _L, C)
    bs = bs.reshape(N, _N_HASHES, _L, 1)
    probs = jax.nn.softmax(bs, axis=1)
    ret = jnp.sum(ret * probs, axis=1)
    ret = ret.transpose(0, 2, 1).reshape(N, C, H, W) + input
    return ret
